# Initial kernel scaffold; baseline (speedup 1.0000x reference)
import jax
import jax.numpy as jnp
from jax.experimental import pallas as pl


def kernel(token_type_ids, table):
    raise NotImplementedError("write your pallas kernel here")


# speedup vs baseline: 8.0636x; 8.0636x over previous
import functools

import jax
import jax.numpy as jnp
from jax import lax
from jax.experimental import pallas as pl
from jax.experimental.pallas import tpu as pltpu
from jax.experimental.pallas import tpu_sc as plsc

L = 16
NS = 16
N_IDS = 4 * 8192
CHUNK = N_IDS // NS
D = 768

_mesh = plsc.VectorSubcoreMesh(core_axis_name="c", subcore_axis_name="s")


def _lane_sum(vec, scratch_ref):
    lanes = jnp.arange(L, dtype=jnp.int32)
    for stride in (1, 2, 4, 8):
        scratch_ref[...] = vec
        vec = vec + plsc.load_gather(scratch_ref, [lanes ^ stride])
    return vec


def _sc_embed_sum_body(ids_hbm, table_hbm, out_hbm, partial_hbm,
                       ids_v, part_v, all_v, table_v, out_v):
    cid = lax.axis_index("c")
    sid = lax.axis_index("s")

    pltpu.sync_copy(ids_hbm.at[pl.ds(sid * CHUNK, CHUNK)], ids_v)

    def id_step(i, acc):
        return acc + ids_v[pl.ds(i * L, L)]

    acc = lax.fori_loop(0, CHUNK // L, id_step, jnp.zeros((L,), jnp.int32))
    part_v[...] = acc

    @pl.when(cid == 0)
    def _publish():
        pltpu.sync_copy(part_v, partial_hbm.at[pl.ds(sid * L, L)])

    plsc.subcore_barrier()

    @pl.when((sid == 0) & (cid == 0))
    def _finalize():
        pltpu.sync_copy(partial_hbm, all_v)

        def part_step(i, acc):
            return acc + all_v[pl.ds(i * L, L)]

        tot = lax.fori_loop(0, NS, part_step, jnp.zeros((L,), jnp.int32))
        n1 = _lane_sum(tot, part_v).astype(jnp.float32)

        pltpu.sync_copy(table_hbm, table_v)

        def row_step(j, carry):
            r0, r1 = carry
            return (r0 + table_v[pl.ds(j * L, L)],
                    r1 + table_v[pl.ds(D + j * L, L)])

        r0, r1 = lax.fori_loop(
            0, D // L, row_step,
            (jnp.zeros((L,), jnp.float32), jnp.zeros((L,), jnp.float32)))
        r0s = _lane_sum(r0, out_v)
        r1s = _lane_sum(r1, out_v)
        res = (jnp.float32(N_IDS) - n1) * r0s + n1 * r1s
        out_v[...] = res
        pltpu.sync_copy(out_v, out_hbm)


_sc_embed_sum = functools.partial(
    pl.kernel,
    mesh=_mesh,
    out_type=(jax.ShapeDtypeStruct((L,), jnp.float32),
              jax.ShapeDtypeStruct((NS * L,), jnp.int32)),
    compiler_params=pltpu.CompilerParams(needs_layout_passes=False),
    scratch_types=[
        pltpu.VMEM((CHUNK,), jnp.int32),
        pltpu.VMEM((L,), jnp.int32),
        pltpu.VMEM((NS * L,), jnp.int32),
        pltpu.VMEM((2 * D,), jnp.float32),
        pltpu.VMEM((L,), jnp.float32),
    ],
)(_sc_embed_sum_body)


def kernel(token_type_ids, table):
    ids = token_type_ids.reshape(-1).astype(jnp.int32)
    res, _ = _sc_embed_sum(ids, table.reshape(-1))
    return res[0]


# speedup vs baseline: 8.7218x; 1.0816x over previous
import functools

import jax
import jax.numpy as jnp
from jax import lax
from jax.experimental import pallas as pl
from jax.experimental.pallas import tpu as pltpu
from jax.experimental.pallas import tpu_sc as plsc

L = 16
NS = 16
N_IDS = 4 * 8192
CHUNK = N_IDS // NS
D = 768

_mesh = plsc.VectorSubcoreMesh(core_axis_name="c", subcore_axis_name="s",
                               num_cores=1)


def _lane_sum(vec, scratch_ref):
    lanes = jnp.arange(L, dtype=jnp.int32)
    for stride in (1, 2, 4, 8):
        scratch_ref[...] = vec
        vec = vec + plsc.load_gather(scratch_ref, [lanes ^ stride])
    return vec


def _sc_embed_sum_body(ids_hbm, table_hbm, out_hbm, parts_hbm, rsum_hbm,
                       ids_v, part_v, all_v, tab_v, rv_v, fv_v,
                       sem_i, sem_t):
    sid = lax.axis_index("s")

    cp_ids = pltpu.async_copy(ids_hbm.at[pl.ds(sid * CHUNK, CHUNK)],
                              ids_v, sem_i)

    def _row_sum(row):
        pltpu.async_copy(table_hbm.at[pl.ds(row * D, D)], tab_v,
                         sem_t).wait()
        r_a = tab_v[pl.ds(0, L)]
        r_b = tab_v[pl.ds(L, L)]
        for j in range(2, D // L, 2):
            r_a = r_a + tab_v[pl.ds(j * L, L)]
            r_b = r_b + tab_v[pl.ds((j + 1) * L, L)]
        rs = _lane_sum(r_a + r_b, fv_v)
        fv_v[...] = rs
        pltpu.sync_copy(fv_v, rsum_hbm.at[pl.ds(row * L, L)])

    @pl.when(sid == 1)
    def _table_row0():
        _row_sum(0)

    @pl.when(sid == 2)
    def _table_row1():
        _row_sum(1)

    cp_ids.wait()
    a0 = ids_v[pl.ds(0, L)]
    a1 = ids_v[pl.ds(L, L)]
    for i in range(2, CHUNK // L, 2):
        a0 = a0 + ids_v[pl.ds(i * L, L)]
        a1 = a1 + ids_v[pl.ds((i + 1) * L, L)]
    part_v[...] = a0 + a1
    pltpu.sync_copy(part_v, parts_hbm.at[pl.ds(sid * L, L)])

    plsc.subcore_barrier()

    @pl.when(sid == 0)
    def _finalize():
        pltpu.sync_copy(parts_hbm, all_v)
        t0 = all_v[pl.ds(0, L)]
        t1 = all_v[pl.ds(L, L)]
        for i in range(2, NS, 2):
            t0 = t0 + all_v[pl.ds(i * L, L)]
            t1 = t1 + all_v[pl.ds((i + 1) * L, L)]
        n1 = _lane_sum(t0 + t1, part_v).astype(jnp.float32)
        pltpu.sync_copy(rsum_hbm, rv_v)
        r0s = rv_v[pl.ds(0, L)]
        r1s = rv_v[pl.ds(L, L)]
        res = (jnp.float32(N_IDS) - n1) * r0s + n1 * r1s
        fv_v[...] = res
        pltpu.sync_copy(fv_v, out_hbm)


_sc_embed_sum = functools.partial(
    pl.kernel,
    mesh=_mesh,
    out_type=(jax.ShapeDtypeStruct((L,), jnp.float32),
              jax.ShapeDtypeStruct((NS * L,), jnp.int32),
              jax.ShapeDtypeStruct((2 * L,), jnp.float32)),
    compiler_params=pltpu.CompilerParams(needs_layout_passes=False),
    scratch_types=[
        pltpu.VMEM((CHUNK,), jnp.int32),
        pltpu.VMEM((L,), jnp.int32),
        pltpu.VMEM((NS * L,), jnp.int32),
        pltpu.VMEM((D,), jnp.float32),
        pltpu.VMEM((2 * L,), jnp.float32),
        pltpu.VMEM((L,), jnp.float32),
        pltpu.SemaphoreType.DMA,
        pltpu.SemaphoreType.DMA,
    ],
)(_sc_embed_sum_body)


def kernel(token_type_ids, table):
    ids = token_type_ids.reshape(-1).astype(jnp.int32)
    res, _, _ = _sc_embed_sum(ids, table.reshape(-1))
    return res[0]


# speedup vs baseline: 8.8934x; 1.0197x over previous
import functools

import jax
import jax.numpy as jnp
from jax import lax
from jax.experimental import pallas as pl
from jax.experimental.pallas import tpu as pltpu
from jax.experimental.pallas import tpu_sc as plsc

L = 16
NS = 16
N_IDS = 4 * 8192
CHUNK = N_IDS // NS
D = 768
SEG = D // 3
NP = NS + 6

_mesh = plsc.VectorSubcoreMesh(core_axis_name="c", subcore_axis_name="s",
                               num_cores=1)


def _lane_sum(vec, scratch_ref):
    lanes = jnp.arange(L, dtype=jnp.int32)
    for stride in (1, 2, 4, 8):
        scratch_ref[...] = vec
        vec = vec + plsc.load_gather(scratch_ref, [lanes ^ stride])
    return vec


def _sc_embed_sum_body(ids_hbm, table_hbm, out_hbm, comb_hbm,
                       ids_v, part_v, all_v, tab_v, fv_v, sem_i, sem_t):
    sid = lax.axis_index("s")

    cp_ids = pltpu.async_copy(ids_hbm.at[pl.ds(sid * CHUNK, CHUNK)],
                              ids_v, sem_i)

    for k in range(6):
        @pl.when(sid == 10 + k)
        def _table_seg(k=k):
            pltpu.async_copy(table_hbm.at[pl.ds(k * SEG, SEG)], tab_v,
                             sem_t).wait()
            r_a = tab_v[pl.ds(0, L)]
            r_b = tab_v[pl.ds(L, L)]
            for j in range(2, SEG // L, 2):
                r_a = r_a + tab_v[pl.ds(j * L, L)]
                r_b = r_b + tab_v[pl.ds((j + 1) * L, L)]
            part_v[...] = plsc.bitcast(r_a + r_b, jnp.int32)
            pltpu.sync_copy(part_v, comb_hbm.at[pl.ds((NS + k) * L, L)])

    cp_ids.wait()
    a0 = ids_v[pl.ds(0, L)]
    a1 = ids_v[pl.ds(L, L)]
    for i in range(2, CHUNK // L, 2):
        a0 = a0 + ids_v[pl.ds(i * L, L)]
        a1 = a1 + ids_v[pl.ds((i + 1) * L, L)]
    part_v[...] = a0 + a1
    pltpu.sync_copy(part_v, comb_hbm.at[pl.ds(sid * L, L)])

    plsc.subcore_barrier()

    @pl.when(sid == 0)
    def _finalize():
        pltpu.sync_copy(comb_hbm, all_v)
        t0 = all_v[pl.ds(0, L)]
        t1 = all_v[pl.ds(L, L)]
        for i in range(2, NS, 2):
            t0 = t0 + all_v[pl.ds(i * L, L)]
            t1 = t1 + all_v[pl.ds((i + 1) * L, L)]
        n1 = _lane_sum(t0 + t1, part_v).astype(jnp.float32)

        def seg(k):
            return plsc.bitcast(all_v[pl.ds((NS + k) * L, L)], jnp.float32)

        r0s = _lane_sum(seg(0) + seg(1) + seg(2), fv_v)
        r1s = _lane_sum(seg(3) + seg(4) + seg(5), fv_v)
        res = (jnp.float32(N_IDS) - n1) * r0s + n1 * r1s
        fv_v[...] = res
        pltpu.sync_copy(fv_v, out_hbm)


_sc_embed_sum = functools.partial(
    pl.kernel,
    mesh=_mesh,
    out_type=(jax.ShapeDtypeStruct((L,), jnp.float32),
              jax.ShapeDtypeStruct((NP * L,), jnp.int32)),
    compiler_params=pltpu.CompilerParams(needs_layout_passes=False),
    scratch_types=[
        pltpu.VMEM((CHUNK,), jnp.int32),
        pltpu.VMEM((L,), jnp.int32),
        pltpu.VMEM((NP * L,), jnp.int32),
        pltpu.VMEM((SEG,), jnp.float32),
        pltpu.VMEM((L,), jnp.float32),
        pltpu.SemaphoreType.DMA,
        pltpu.SemaphoreType.DMA,
    ],
)(_sc_embed_sum_body)


def kernel(token_type_ids, table):
    ids = token_type_ids.reshape(-1).astype(jnp.int32)
    res, _ = _sc_embed_sum(ids, table.reshape(-1))
    return res[0]


# speedup vs baseline: 8.9068x; 1.0015x over previous
import functools

import jax
import jax.numpy as jnp
from jax import lax
from jax.experimental import pallas as pl
from jax.experimental.pallas import tpu as pltpu
from jax.experimental.pallas import tpu_sc as plsc

L = 16
NS = 16
N_IDS = 4 * 8192
CHUNK = N_IDS // NS
D = 768
SEG = D // 3
NP = NS + 6

_mesh = plsc.VectorSubcoreMesh(core_axis_name="c", subcore_axis_name="s",
                               num_cores=1)


def _lane_sum(vec, scratch_ref):
    lanes = jnp.arange(L, dtype=jnp.int32)
    for stride in (1, 2, 4, 8):
        scratch_ref[...] = vec
        vec = vec + plsc.load_gather(scratch_ref, [lanes ^ stride])
    return vec


def _sc_embed_sum_body(ids_hbm, table_hbm, out_hbm, comb_hbm,
                       ids_v, part_v, all_v, tab_v, fv_v, sem_i, sem_t):
    sid = lax.axis_index("s")

    cp_ids = pltpu.async_copy(ids_hbm.at[pl.ds(sid * CHUNK, CHUNK)],
                              ids_v, sem_i)

    for k in range(6):
        @pl.when(sid == 10 + k)
        def _table_seg(k=k):
            pltpu.async_copy(table_hbm.at[pl.ds(k * SEG, SEG)], tab_v,
                             sem_t).wait()
            r_a = tab_v[pl.ds(0, L)]
            r_b = tab_v[pl.ds(L, L)]
            for j in range(2, SEG // L, 2):
                r_a = r_a + tab_v[pl.ds(j * L, L)]
                r_b = r_b + tab_v[pl.ds((j + 1) * L, L)]
            rs = _lane_sum(r_a + r_b, fv_v)
            part_v[...] = plsc.bitcast(rs, jnp.int32)
            pltpu.sync_copy(part_v, comb_hbm.at[pl.ds((NS + k) * L, L)])

    cp_ids.wait()
    a0 = ids_v[pl.ds(0, L)]
    a1 = ids_v[pl.ds(L, L)]
    for i in range(2, CHUNK // L, 2):
        a0 = a0 + ids_v[pl.ds(i * L, L)]
        a1 = a1 + ids_v[pl.ds((i + 1) * L, L)]
    part_v[...] = _lane_sum(a0 + a1, part_v)
    pltpu.sync_copy(part_v, comb_hbm.at[pl.ds(sid * L, L)])

    plsc.subcore_barrier()

    @pl.when(sid == 0)
    def _finalize():
        pltpu.sync_copy(comb_hbm, all_v)
        t0 = all_v[pl.ds(0, L)]
        t1 = all_v[pl.ds(L, L)]
        for i in range(2, NS, 2):
            t0 = t0 + all_v[pl.ds(i * L, L)]
            t1 = t1 + all_v[pl.ds((i + 1) * L, L)]
        n1 = (t0 + t1).astype(jnp.float32)

        def seg(k):
            return plsc.bitcast(all_v[pl.ds((NS + k) * L, L)], jnp.float32)

        r0s = seg(0) + seg(1) + seg(2)
        r1s = seg(3) + seg(4) + seg(5)
        res = (jnp.float32(N_IDS) - n1) * r0s + n1 * r1s
        fv_v[...] = res
        pltpu.sync_copy(fv_v, out_hbm)


_sc_embed_sum = functools.partial(
    pl.kernel,
    mesh=_mesh,
    out_type=(jax.ShapeDtypeStruct((L,), jnp.float32),
              jax.ShapeDtypeStruct((NP * L,), jnp.int32)),
    compiler_params=pltpu.CompilerParams(needs_layout_passes=False),
    scratch_types=[
        pltpu.VMEM((CHUNK,), jnp.int32),
        pltpu.VMEM((L,), jnp.int32),
        pltpu.VMEM((NP * L,), jnp.int32),
        pltpu.VMEM((SEG,), jnp.float32),
        pltpu.VMEM((L,), jnp.float32),
        pltpu.SemaphoreType.DMA,
        pltpu.SemaphoreType.DMA,
    ],
)(_sc_embed_sum_body)


def kernel(token_type_ids, table):
    ids = token_type_ids.reshape(-1).astype(jnp.int32)
    res, _ = _sc_embed_sum(ids, table.reshape(-1))
    return res[0]
